# baseline (speedup 1.0000x reference)
import jax
import jax.numpy as jnp
from jax.experimental import pallas as pl
from jax.experimental.pallas import tpu as pltpu

N = 320000
DF = 128
BN = 256
NB = N // BN
CV = 3 * DF
HP = jax.lax.Precision.HIGHEST


def _dot(a, b):
    return jax.lax.dot_general(a, b, (((1,), (0,)), ((), ())),
                               precision=HP,
                               preferred_element_type=jnp.float32)


def _payload(h, e, u):
    uh = u * h
    uh2 = uh * h
    col = jax.lax.broadcasted_iota(jnp.int32, (BN, DF), 1)
    extra = jnp.where(col == 0, e, 0.0) + jnp.where(col == 1, u, 0.0)
    return jnp.concatenate([uh, uh2, extra], axis=1)


def _fwd_kernel(x_ref, idx_ref, wp_ref, wct_ref, bc_ref, wa_ref, ba_ref,
                h_ref, e_ref, p_ref, cseg_ref, cval_ref):
    i = pl.program_id(0)

    @pl.when(i == 0)
    def _():
        cseg_ref[0, 0] = -1
        cval_ref[...] = jnp.zeros((1, CV), jnp.float32)

    x = x_ref[...]
    h = _dot(x, wct_ref[...]) + bc_ref[...]
    att = _dot(h, wa_ref[...]) + ba_ref[0, 0]
    e = jnp.exp(att)
    u = e * wp_ref[...]
    acc = _payload(h, e, u)
    sid = idx_ref[...]

    d = 1
    while d < BN:
        accs = jnp.concatenate(
            [jnp.zeros((d, CV), jnp.float32), acc[:BN - d]], axis=0)
        sids = jnp.concatenate(
            [jnp.full((d, 1), -1, jnp.int32), sid[:BN - d]], axis=0)
        acc = acc + jnp.where(sids == sid, accs, 0.0)
        d *= 2

    acc = acc + jnp.where(sid == cseg_ref[0, 0], cval_ref[...], 0.0)

    h_ref[...] = h
    e_ref[...] = e
    p_ref[...] = acc
    cseg_ref[0, 0] = idx_ref[BN - 1, 0]
    cval_ref[...] = acc[BN - 1:BN, :]


def _bwd_kernel(p_ref, idx_ref, h_ref, e_ref, wp_ref, mg_ref, gg_ref, gb_ref,
                out_ref, ra_ref, bseg_ref, bT_ref):
    i = pl.program_id(0)

    @pl.when(i == 0)
    def _():
        bseg_ref[0, 0] = -2
        bT_ref[...] = jnp.zeros((1, CV), jnp.float32)

    sid = idx_ref[...]
    P = p_ref[...]
    h = h_ref[...]
    e = e_ref[...]
    u = e * wp_ref[...]
    V = _payload(h, e, u)

    R = V
    d = 1
    while d < BN:
        Rs = jnp.concatenate(
            [R[d:], jnp.zeros((d, CV), jnp.float32)], axis=0)
        ss = jnp.concatenate(
            [sid[d:], jnp.full((d, 1), -3, jnp.int32)], axis=0)
        R = R + jnp.where(ss == sid, Rs, 0.0)
        d *= 2

    T = P + R - V
    fut = bT_ref[...] - P[BN - 1:BN, :]
    T = T + jnp.where(sid == bseg_ref[0, 0], fut, 0.0)

    bseg_ref[0, 0] = idx_ref[0, 0]
    bT_ref[...] = T[0:1, :]

    Sh = T[:, 0:DF]
    Sh2 = T[:, DF:2 * DF]
    E = T[:, 2 * DF:2 * DF + 1]
    U = T[:, 2 * DF + 1:2 * DF + 2]
    D = jnp.maximum(U, 0.001 * E)
    a = u / D
    c = U / D
    mean = Sh / D
    var = Sh2 / D - (2.0 - c) * (mean * mean)
    std = jnp.sqrt(var + 0.001)
    outn = (h - mean) / std

    mg = mg_ref[...]
    gs = _dot(outn, mg) * 0.25
    gss = _dot(outn * outn, mg) * 0.25
    gvar = gss - gs * gs
    og = (outn - gs) * jax.lax.rsqrt(gvar + 1e-5)
    out = og * gg_ref[...] + gb_ref[...]
    out_ref[...] = jnp.maximum(out, 0.0)
    ra_ref[...] = a


@jax.jit
def kernel(x, segment_idx, weight_pri, W_conv, b_conv, W_att, b_att,
           gn_gamma, gn_beta):
    idx = segment_idx.astype(jnp.int32).reshape(N, 1)
    wp = weight_pri.reshape(N, 1)
    wct = W_conv.T
    bc = b_conv.reshape(1, DF)
    wa = W_att.reshape(DF, 1)
    ba = b_att.reshape(1, 1)
    gg = gn_gamma.reshape(1, DF)
    gb = gn_beta.reshape(1, DF)
    gidx = jnp.arange(DF) // 4
    mg = (gidx[:, None] == gidx[None, :]).astype(jnp.float32)

    row = lambda i: (i, 0)
    rep = lambda i: (0, 0)

    h, e, P = pl.pallas_call(
        _fwd_kernel,
        grid=(NB,),
        in_specs=[
            pl.BlockSpec((BN, DF), row),
            pl.BlockSpec((BN, 1), row),
            pl.BlockSpec((BN, 1), row),
            pl.BlockSpec((DF, DF), rep),
            pl.BlockSpec((1, DF), rep),
            pl.BlockSpec((DF, 1), rep),
            pl.BlockSpec((1, 1), rep),
        ],
        out_specs=[
            pl.BlockSpec((BN, DF), row),
            pl.BlockSpec((BN, 1), row),
            pl.BlockSpec((BN, CV), row),
        ],
        out_shape=[
            jax.ShapeDtypeStruct((N, DF), jnp.float32),
            jax.ShapeDtypeStruct((N, 1), jnp.float32),
            jax.ShapeDtypeStruct((N, CV), jnp.float32),
        ],
        scratch_shapes=[
            pltpu.SMEM((1, 1), jnp.int32),
            pltpu.VMEM((1, CV), jnp.float32),
        ],
    )(x, idx, wp, wct, bc, wa, ba)

    rev = lambda i: (NB - 1 - i, 0)
    out, ra = pl.pallas_call(
        _bwd_kernel,
        grid=(NB,),
        in_specs=[
            pl.BlockSpec((BN, CV), rev),
            pl.BlockSpec((BN, 1), rev),
            pl.BlockSpec((BN, DF), rev),
            pl.BlockSpec((BN, 1), rev),
            pl.BlockSpec((BN, 1), rev),
            pl.BlockSpec((DF, DF), rep),
            pl.BlockSpec((1, DF), rep),
            pl.BlockSpec((1, DF), rep),
        ],
        out_specs=[
            pl.BlockSpec((BN, DF), rev),
            pl.BlockSpec((BN, 1), rev),
        ],
        out_shape=[
            jax.ShapeDtypeStruct((N, DF), jnp.float32),
            jax.ShapeDtypeStruct((N, 1), jnp.float32),
        ],
        scratch_shapes=[
            pltpu.SMEM((1, 1), jnp.int32),
            pltpu.VMEM((1, CV), jnp.float32),
        ],
    )(P, idx, h, e, wp, mg, gg, gb)

    return out, ra


# speedup vs baseline: 1.2108x; 1.2108x over previous
import jax
import jax.numpy as jnp
from jax.experimental import pallas as pl
from jax.experimental.pallas import tpu as pltpu

N = 320000
DF = 128
BN = 512
NB = N // BN
CV = 3 * DF
HP = jax.lax.Precision.HIGHEST


def _dot(a, b):
    return jax.lax.dot_general(a, b, (((1,), (0,)), ((), ())),
                               precision=HP,
                               preferred_element_type=jnp.float32)


def _payload(h, e, u):
    uh = u * h
    uh2 = uh * h
    col = jax.lax.broadcasted_iota(jnp.int32, (BN, DF), 1)
    extra = jnp.where(col == 0, e, 0.0) + jnp.where(col == 1, u, 0.0)
    return jnp.concatenate([uh, uh2, extra], axis=1)


def _fwd_kernel(x_ref, idx_ref, wp_ref, wct_ref, bc_ref, wa_ref, ba_ref,
                h_ref, e_ref, p_ref, cseg_ref, cval_ref):
    i = pl.program_id(0)

    @pl.when(i == 0)
    def _():
        cseg_ref[0, 0] = -1
        cval_ref[...] = jnp.zeros((1, CV), jnp.float32)

    x = x_ref[...]
    h = _dot(x, wct_ref[...]) + bc_ref[...]
    att = _dot(h, wa_ref[...]) + ba_ref[0, 0]
    e = jnp.exp(att)
    u = e * wp_ref[...]
    acc = _payload(h, e, u)
    sid = idx_ref[...]

    d = 1
    while d < BN:
        accs = jnp.concatenate(
            [jnp.zeros((d, CV), jnp.float32), acc[:BN - d]], axis=0)
        sids = jnp.concatenate(
            [jnp.full((d, 1), -1, jnp.int32), sid[:BN - d]], axis=0)
        acc = acc + jnp.where(sids == sid, accs, 0.0)
        d *= 2

    acc = acc + jnp.where(sid == cseg_ref[0, 0], cval_ref[...], 0.0)

    h_ref[...] = h
    e_ref[...] = e
    p_ref[...] = acc
    cseg_ref[0, 0] = idx_ref[BN - 1, 0]
    cval_ref[...] = acc[BN - 1:BN, :]


def _bwd_kernel(p_ref, idx_ref, h_ref, e_ref, wp_ref, mg_ref, gg_ref, gb_ref,
                out_ref, ra_ref, bseg_ref, bT_ref):
    i = pl.program_id(0)

    @pl.when(i == 0)
    def _():
        bseg_ref[0, 0] = -2
        bT_ref[...] = jnp.zeros((1, CV), jnp.float32)

    sid = idx_ref[...]
    P = p_ref[...]
    h = h_ref[...]
    e = e_ref[...]
    u = e * wp_ref[...]
    V = _payload(h, e, u)

    R = V
    d = 1
    while d < BN:
        Rs = jnp.concatenate(
            [R[d:], jnp.zeros((d, CV), jnp.float32)], axis=0)
        ss = jnp.concatenate(
            [sid[d:], jnp.full((d, 1), -3, jnp.int32)], axis=0)
        R = R + jnp.where(ss == sid, Rs, 0.0)
        d *= 2

    T = P + R - V
    fut = bT_ref[...] - P[BN - 1:BN, :]
    T = T + jnp.where(sid == bseg_ref[0, 0], fut, 0.0)

    bseg_ref[0, 0] = idx_ref[0, 0]
    bT_ref[...] = T[0:1, :]

    Sh = T[:, 0:DF]
    Sh2 = T[:, DF:2 * DF]
    E = T[:, 2 * DF:2 * DF + 1]
    U = T[:, 2 * DF + 1:2 * DF + 2]
    D = jnp.maximum(U, 0.001 * E)
    a = u / D
    c = U / D
    mean = Sh / D
    var = Sh2 / D - (2.0 - c) * (mean * mean)
    std = jnp.sqrt(var + 0.001)
    outn = (h - mean) / std

    mg = mg_ref[...]
    gs = _dot(outn, mg) * 0.25
    gss = _dot(outn * outn, mg) * 0.25
    gvar = gss - gs * gs
    og = (outn - gs) * jax.lax.rsqrt(gvar + 1e-5)
    out = og * gg_ref[...] + gb_ref[...]
    out_ref[...] = jnp.maximum(out, 0.0)
    ra_ref[...] = a


@jax.jit
def kernel(x, segment_idx, weight_pri, W_conv, b_conv, W_att, b_att,
           gn_gamma, gn_beta):
    idx = segment_idx.astype(jnp.int32).reshape(N, 1)
    wp = weight_pri.reshape(N, 1)
    wct = W_conv.T
    bc = b_conv.reshape(1, DF)
    wa = W_att.reshape(DF, 1)
    ba = b_att.reshape(1, 1)
    gg = gn_gamma.reshape(1, DF)
    gb = gn_beta.reshape(1, DF)
    gidx = jnp.arange(DF) // 4
    mg = (gidx[:, None] == gidx[None, :]).astype(jnp.float32)

    row = lambda i: (i, 0)
    rep = lambda i: (0, 0)

    h, e, P = pl.pallas_call(
        _fwd_kernel,
        grid=(NB,),
        in_specs=[
            pl.BlockSpec((BN, DF), row),
            pl.BlockSpec((BN, 1), row),
            pl.BlockSpec((BN, 1), row),
            pl.BlockSpec((DF, DF), rep),
            pl.BlockSpec((1, DF), rep),
            pl.BlockSpec((DF, 1), rep),
            pl.BlockSpec((1, 1), rep),
        ],
        out_specs=[
            pl.BlockSpec((BN, DF), row),
            pl.BlockSpec((BN, 1), row),
            pl.BlockSpec((BN, CV), row),
        ],
        out_shape=[
            jax.ShapeDtypeStruct((N, DF), jnp.float32),
            jax.ShapeDtypeStruct((N, 1), jnp.float32),
            jax.ShapeDtypeStruct((N, CV), jnp.float32),
        ],
        scratch_shapes=[
            pltpu.SMEM((1, 1), jnp.int32),
            pltpu.VMEM((1, CV), jnp.float32),
        ],
    )(x, idx, wp, wct, bc, wa, ba)

    rev = lambda i: (NB - 1 - i, 0)
    out, ra = pl.pallas_call(
        _bwd_kernel,
        grid=(NB,),
        in_specs=[
            pl.BlockSpec((BN, CV), rev),
            pl.BlockSpec((BN, 1), rev),
            pl.BlockSpec((BN, DF), rev),
            pl.BlockSpec((BN, 1), rev),
            pl.BlockSpec((BN, 1), rev),
            pl.BlockSpec((DF, DF), rep),
            pl.BlockSpec((1, DF), rep),
            pl.BlockSpec((1, DF), rep),
        ],
        out_specs=[
            pl.BlockSpec((BN, DF), rev),
            pl.BlockSpec((BN, 1), rev),
        ],
        out_shape=[
            jax.ShapeDtypeStruct((N, DF), jnp.float32),
            jax.ShapeDtypeStruct((N, 1), jnp.float32),
        ],
        scratch_shapes=[
            pltpu.SMEM((1, 1), jnp.int32),
            pltpu.VMEM((1, CV), jnp.float32),
        ],
    )(P, idx, h, e, wp, mg, gg, gb)

    return out, ra


# speedup vs baseline: 1.2293x; 1.0152x over previous
import jax
import jax.numpy as jnp
from jax.experimental import pallas as pl
from jax.experimental.pallas import tpu as pltpu

N = 320000
DF = 128
BN = 640
NB = N // BN
CV = 3 * DF
HP = jax.lax.Precision.HIGHEST


def _dot(a, b):
    return jax.lax.dot_general(a, b, (((1,), (0,)), ((), ())),
                               precision=HP,
                               preferred_element_type=jnp.float32)


def _payload(h, e, u):
    uh = u * h
    uh2 = uh * h
    col = jax.lax.broadcasted_iota(jnp.int32, (BN, DF), 1)
    extra = jnp.where(col == 0, e, 0.0) + jnp.where(col == 1, u, 0.0)
    return jnp.concatenate([uh, uh2, extra], axis=1)


def _fwd_kernel(x_ref, idx_ref, wp_ref, wct_ref, bc_ref, wa_ref, ba_ref,
                h_ref, e_ref, p_ref, cseg_ref, cval_ref):
    i = pl.program_id(0)

    @pl.when(i == 0)
    def _():
        cseg_ref[0, 0] = -1
        cval_ref[...] = jnp.zeros((1, CV), jnp.float32)

    x = x_ref[...]
    h = _dot(x, wct_ref[...]) + bc_ref[...]
    att = _dot(h, wa_ref[...]) + ba_ref[0, 0]
    e = jnp.exp(att)
    u = e * wp_ref[...]
    acc = _payload(h, e, u)
    sid = idx_ref[...]

    d = 1
    while d < BN:
        accs = jnp.concatenate(
            [jnp.zeros((d, CV), jnp.float32), acc[:BN - d]], axis=0)
        sids = jnp.concatenate(
            [jnp.full((d, 1), -1, jnp.int32), sid[:BN - d]], axis=0)
        acc = acc + jnp.where(sids == sid, accs, 0.0)
        d *= 2

    acc = acc + jnp.where(sid == cseg_ref[0, 0], cval_ref[...], 0.0)

    h_ref[...] = h
    e_ref[...] = e
    p_ref[...] = acc
    cseg_ref[0, 0] = idx_ref[BN - 1, 0]
    cval_ref[...] = acc[BN - 1:BN, :]


def _bwd_kernel(p_ref, idx_ref, h_ref, e_ref, wp_ref, mg_ref, gg_ref, gb_ref,
                out_ref, ra_ref, bseg_ref, bT_ref):
    i = pl.program_id(0)

    @pl.when(i == 0)
    def _():
        bseg_ref[0, 0] = -2
        bT_ref[...] = jnp.zeros((1, CV), jnp.float32)

    sid = idx_ref[...]
    P = p_ref[...]
    h = h_ref[...]
    e = e_ref[...]
    u = e * wp_ref[...]
    V = _payload(h, e, u)

    R = V
    d = 1
    while d < BN:
        Rs = jnp.concatenate(
            [R[d:], jnp.zeros((d, CV), jnp.float32)], axis=0)
        ss = jnp.concatenate(
            [sid[d:], jnp.full((d, 1), -3, jnp.int32)], axis=0)
        R = R + jnp.where(ss == sid, Rs, 0.0)
        d *= 2

    T = P + R - V
    fut = bT_ref[...] - P[BN - 1:BN, :]
    T = T + jnp.where(sid == bseg_ref[0, 0], fut, 0.0)

    bseg_ref[0, 0] = idx_ref[0, 0]
    bT_ref[...] = T[0:1, :]

    Sh = T[:, 0:DF]
    Sh2 = T[:, DF:2 * DF]
    E = T[:, 2 * DF:2 * DF + 1]
    U = T[:, 2 * DF + 1:2 * DF + 2]
    D = jnp.maximum(U, 0.001 * E)
    a = u / D
    c = U / D
    mean = Sh / D
    var = Sh2 / D - (2.0 - c) * (mean * mean)
    std = jnp.sqrt(var + 0.001)
    outn = (h - mean) / std

    mg = mg_ref[...]
    gs = _dot(outn, mg) * 0.25
    gss = _dot(outn * outn, mg) * 0.25
    gvar = gss - gs * gs
    og = (outn - gs) * jax.lax.rsqrt(gvar + 1e-5)
    out = og * gg_ref[...] + gb_ref[...]
    out_ref[...] = jnp.maximum(out, 0.0)
    ra_ref[...] = a


@jax.jit
def kernel(x, segment_idx, weight_pri, W_conv, b_conv, W_att, b_att,
           gn_gamma, gn_beta):
    idx = segment_idx.astype(jnp.int32).reshape(N, 1)
    wp = weight_pri.reshape(N, 1)
    wct = W_conv.T
    bc = b_conv.reshape(1, DF)
    wa = W_att.reshape(DF, 1)
    ba = b_att.reshape(1, 1)
    gg = gn_gamma.reshape(1, DF)
    gb = gn_beta.reshape(1, DF)
    gidx = jnp.arange(DF) // 4
    mg = (gidx[:, None] == gidx[None, :]).astype(jnp.float32)

    row = lambda i: (i, 0)
    rep = lambda i: (0, 0)

    h, e, P = pl.pallas_call(
        _fwd_kernel,
        grid=(NB,),
        in_specs=[
            pl.BlockSpec((BN, DF), row),
            pl.BlockSpec((BN, 1), row),
            pl.BlockSpec((BN, 1), row),
            pl.BlockSpec((DF, DF), rep),
            pl.BlockSpec((1, DF), rep),
            pl.BlockSpec((DF, 1), rep),
            pl.BlockSpec((1, 1), rep),
        ],
        out_specs=[
            pl.BlockSpec((BN, DF), row),
            pl.BlockSpec((BN, 1), row),
            pl.BlockSpec((BN, CV), row),
        ],
        out_shape=[
            jax.ShapeDtypeStruct((N, DF), jnp.float32),
            jax.ShapeDtypeStruct((N, 1), jnp.float32),
            jax.ShapeDtypeStruct((N, CV), jnp.float32),
        ],
        scratch_shapes=[
            pltpu.SMEM((1, 1), jnp.int32),
            pltpu.VMEM((1, CV), jnp.float32),
        ],
    )(x, idx, wp, wct, bc, wa, ba)

    rev = lambda i: (NB - 1 - i, 0)
    out, ra = pl.pallas_call(
        _bwd_kernel,
        grid=(NB,),
        in_specs=[
            pl.BlockSpec((BN, CV), rev),
            pl.BlockSpec((BN, 1), rev),
            pl.BlockSpec((BN, DF), rev),
            pl.BlockSpec((BN, 1), rev),
            pl.BlockSpec((BN, 1), rev),
            pl.BlockSpec((DF, DF), rep),
            pl.BlockSpec((1, DF), rep),
            pl.BlockSpec((1, DF), rep),
        ],
        out_specs=[
            pl.BlockSpec((BN, DF), rev),
            pl.BlockSpec((BN, 1), rev),
        ],
        out_shape=[
            jax.ShapeDtypeStruct((N, DF), jnp.float32),
            jax.ShapeDtypeStruct((N, 1), jnp.float32),
        ],
        scratch_shapes=[
            pltpu.SMEM((1, 1), jnp.int32),
            pltpu.VMEM((1, CV), jnp.float32),
        ],
    )(P, idx, h, e, wp, mg, gg, gb)

    return out, ra
